# baseline (speedup 1.0000x reference)
import functools

import jax
import jax.numpy as jnp
from jax import lax
from jax.experimental import pallas as pl
from jax.experimental.pallas import tpu as pltpu
from jax.experimental.pallas import tpu_sc as plsc

N_IN = 2048
D_IN = 64
D_HID = 256
N_OUT = 10000
ROWS = 1000

SC_CORES = 2
SC_TILES = 16
N_PAD = 10240
E_RAW = 320000 + N_OUT
EROWS = 162
E_PAD = SC_TILES * EROWS * 128
PAD_DST = N_OUT + 16


def _leaky(v):
    return jnp.where(v >= 0, v, 0.2 * v)


def _style_body(s1, s2, w11, w12, w13, w14, b11, b12, b13, b14,
                w21, w22, w23, w24, b21, b22, b23, b24,
                w31, w32, w33, w34, b31, b32, b33, b34,
                ga1, bb1, ga2, bb2, ga3, bb3):
    def mm(s, w):
        return lax.dot_general(s[...], w[...], (((1,), (1,)), ((), ())),
                               preferred_element_type=jnp.float32)
    ga1[...] = 0.5 * (mm(s1, w11) + b11[...]) + 0.5 * (mm(s2, w13) + b13[...])
    bb1[...] = 0.5 * (mm(s1, w12) + b12[...]) + 0.5 * (mm(s2, w14) + b14[...])
    ga2[...] = 0.5 * (mm(s1, w21) + b21[...]) + 0.5 * (mm(s2, w23) + b23[...])
    bb2[...] = 0.5 * (mm(s1, w22) + b22[...]) + 0.5 * (mm(s2, w24) + b24[...])
    ga3[...] = 0.5 * (mm(s1, w31) + b31[...]) + 0.5 * (mm(s2, w33) + b33[...])
    bb3[...] = 0.5 * (mm(s1, w32) + b32[...]) + 0.5 * (mm(s2, w34) + b34[...])


def _styles(s1, s2, p):
    args = [s1, s2]
    for tag in ('a1', 'a2', 'a3'):
        args += [p[tag + '_W1'], p[tag + '_W2'], p[tag + '_W3'], p[tag + '_W4'],
                 p[tag + '_b1'].reshape(1, -1), p[tag + '_b2'].reshape(1, -1),
                 p[tag + '_b3'].reshape(1, -1), p[tag + '_b4'].reshape(1, -1)]
    outs = [jax.ShapeDtypeStruct((1, 64), jnp.float32),
            jax.ShapeDtypeStruct((1, 64), jnp.float32),
            jax.ShapeDtypeStruct((1, 256), jnp.float32),
            jax.ShapeDtypeStruct((1, 256), jnp.float32),
            jax.ShapeDtypeStruct((1, 64), jnp.float32),
            jax.ShapeDtypeStruct((1, 64), jnp.float32)]
    return pl.pallas_call(_style_body, out_shape=outs)(*args)


def _stage1_body(w_ref, x_ref, trsb_ref, g_ref, b_ref, ga_ref, bb_ref,
                 fw_ref, fb_ref, u_ref, sums_ref):
    i = pl.program_id(0)
    t = lax.dot_general(w_ref[...], x_ref[...], (((1,), (0,)), ((), ())),
                        preferred_element_type=jnp.float32)
    t = t + trsb_ref[...]
    mu = jnp.mean(t, axis=1, keepdims=True)
    xc = t - mu
    var = jnp.mean(xc * xc, axis=1, keepdims=True)
    t = xc * lax.rsqrt(var + 1e-5) * g_ref[...] + b_ref[...]
    t = _leaky(t)
    m2 = jnp.mean(t, axis=1, keepdims=True)
    c2 = t - m2
    std = jnp.sqrt(jnp.sum(c2 * c2, axis=1, keepdims=True) * (1.0 / (D_IN - 1)))
    h = ga_ref[...] * (c2 / (std + 1e-8)) + bb_ref[...]
    u = lax.dot_general(h, fw_ref[...], (((1,), (1,)), ((), ())),
                        preferred_element_type=jnp.float32)
    u = u + fb_ref[...]
    u_ref[...] = u

    @pl.when(i == 0)
    def _():
        sums_ref[...] = jnp.zeros_like(sums_ref)
    sums_ref[0:1, :] += jnp.sum(u, axis=0, keepdims=True)
    sums_ref[1:2, :] += jnp.sum(u * u, axis=0, keepdims=True)


def _stage1(x, p, ga1, bb1):
    nblk = N_OUT // ROWS
    u, sums = pl.pallas_call(
        _stage1_body,
        grid=(nblk,),
        in_specs=[
            pl.BlockSpec((ROWS, N_IN), lambda i: (i, 0)),
            pl.BlockSpec((N_IN, D_IN), lambda i: (0, 0)),
            pl.BlockSpec((ROWS, 1), lambda i: (i, 0)),
            pl.BlockSpec((ROWS, 1), lambda i: (i, 0)),
            pl.BlockSpec((ROWS, 1), lambda i: (i, 0)),
            pl.BlockSpec((1, 64), lambda i: (0, 0)),
            pl.BlockSpec((1, 64), lambda i: (0, 0)),
            pl.BlockSpec((D_HID, D_IN), lambda i: (0, 0)),
            pl.BlockSpec((1, D_HID), lambda i: (0, 0)),
        ],
        out_specs=[
            pl.BlockSpec((ROWS, D_HID), lambda i: (i, 0)),
            pl.BlockSpec((8, D_HID), lambda i: (0, 0)),
        ],
        out_shape=[
            jax.ShapeDtypeStruct((N_OUT, D_HID), jnp.float32),
            jax.ShapeDtypeStruct((8, D_HID), jnp.float32),
        ],
    )(p['trs_W'], x, p['trs_b'].reshape(-1, 1), p['bn2_g'].reshape(-1, 1),
      p['bn2_b'].reshape(-1, 1), ga1, bb1, p['fc1_W'],
      p['fc1_b'].reshape(1, -1))
    return u, sums


def _stage2_body(u_ref, sums_ref, g_ref, b_ref, ga_ref, bb_ref, gw_ref,
                 av_ref, xl_ref, aa_ref):
    u = u_ref[...]
    s = sums_ref[...]
    mu = s[0:1, :] * (1.0 / N_OUT)
    ex2 = s[1:2, :] * (1.0 / N_OUT)
    var = ex2 - mu * mu
    h = (u - mu) * lax.rsqrt(var + 1e-5) * g_ref[...] + b_ref[...]
    h = _leaky(h)
    m = jnp.mean(h, axis=1, keepdims=True)
    c = h - m
    std = jnp.sqrt(jnp.sum(c * c, axis=1, keepdims=True) * (1.0 / (D_HID - 1)))
    h2 = ga_ref[...] * (c / (std + 1e-8)) + bb_ref[...]
    xl = lax.dot_general(h2, gw_ref[...], (((1,), (1,)), ((), ())),
                         preferred_element_type=jnp.float32)
    xl_ref[...] = xl
    aa_ref[...] = jnp.dot(xl, av_ref[...], preferred_element_type=jnp.float32)


def _stage2(u, sums, p, ga2, bb2):
    nblk = N_OUT // ROWS
    av = jnp.stack([p['gat_asrc'], p['gat_adst']], axis=1)
    xl, aa = pl.pallas_call(
        _stage2_body,
        grid=(nblk,),
        in_specs=[
            pl.BlockSpec((ROWS, D_HID), lambda i: (i, 0)),
            pl.BlockSpec((8, D_HID), lambda i: (0, 0)),
            pl.BlockSpec((1, D_HID), lambda i: (0, 0)),
            pl.BlockSpec((1, D_HID), lambda i: (0, 0)),
            pl.BlockSpec((1, D_HID), lambda i: (0, 0)),
            pl.BlockSpec((1, D_HID), lambda i: (0, 0)),
            pl.BlockSpec((D_IN, D_HID), lambda i: (0, 0)),
            pl.BlockSpec((D_IN, 2), lambda i: (0, 0)),
        ],
        out_specs=[
            pl.BlockSpec((ROWS, D_IN), lambda i: (i, 0)),
            pl.BlockSpec((ROWS, 2), lambda i: (i, 0)),
        ],
        out_shape=[
            jax.ShapeDtypeStruct((N_OUT, D_IN), jnp.float32),
            jax.ShapeDtypeStruct((N_OUT, 2), jnp.float32),
        ],
    )(u, sums, p['bn1_g'].reshape(1, -1), p['bn1_b'].reshape(1, -1),
      ga2, bb2, p['gat_W'], av)
    return xl, aa


def _edge1_body(edgeb, asrcp, adstp, wout, denout,
                pk_v, dst_v, w_v, asrc_v, adst_v, den_v, den_sh, sem):
    s = lax.axis_index("s")
    nslice = N_PAD // SC_TILES
    pltpu.sync_copy(edgeb.at[s], pk_v)
    pltpu.sync_copy(asrcp, asrc_v)
    pltpu.sync_copy(adstp, adst_v)

    zv = jnp.zeros((16,), jnp.float32)

    def zrow(j, carry):
        den_v[pl.ds(j * 16, 16)] = zv
        return carry
    lax.fori_loop(0, N_PAD // 16, zrow, 0)
    pltpu.sync_copy(den_v.at[pl.ds(s * nslice, nslice)],
                    den_sh.at[pl.ds(s * nslice, nslice)])

    def p1(j, carry):
        def p1c(k, carry2):
            sl = pl.ds(k * 16, 16)
            v = pk_v[j, sl]
            si = v & 16383
            di = lax.shift_right_logical(v, 14)
            dst_v[j, sl] = di
            e = plsc.load_gather(asrc_v, [si]) + plsc.load_gather(adst_v, [di])
            e = jnp.where(e >= 0, e, 0.2 * e)
            w_v[j, sl] = jnp.exp(e)
            return carry2
        return lax.fori_loop(0, 8, p1c, carry)
    lax.fori_loop(0, EROWS, p1, 0)
    plsc.subcore_barrier()

    def p2(j, carry):
        pltpu.sync_copy(w_v.at[j], den_sh.at[dst_v.at[j]], add=True)
        return carry
    lax.fori_loop(0, EROWS, p2, 0)
    plsc.subcore_barrier()
    pltpu.sync_copy(w_v, wout.at[s])
    pltpu.sync_copy(den_sh.at[pl.ds(s * nslice, nslice)],
                    denout.at[pl.ds(s * nslice, nslice)])


def _edge2_body(edgeb, wh, denh, xlh, out2,
                pk_v, w_v, den_v, rowsA, rowsB, scA, scB, siA, siB,
                diA, diB, out_sh, semA, semB, semSA, semSB):
    s = lax.axis_index("s")
    nslice = N_PAD // SC_TILES
    pltpu.sync_copy(edgeb.at[s], pk_v)
    pltpu.sync_copy(wh.at[s], w_v)
    pltpu.sync_copy(denh, den_v)

    zv = jnp.zeros((16,), jnp.float32)

    def zrow(j, carry):
        for c4 in range(4):
            scA[j, pl.ds(c4 * 16, 16)] = zv
        return carry
    lax.fori_loop(0, 128, zrow, 0)

    def zout(j, carry):
        pltpu.sync_copy(scA, out_sh.at[pl.ds(s * nslice + j * 128, 128)])
        return carry
    lax.fori_loop(0, nslice // 128, zout, 0)
    plsc.subcore_barrier()

    lanes = lax.iota(jnp.int32, 16)

    def fill_si(j, si_b):
        def fs(k, carry):
            sl = pl.ds(k * 16, 16)
            si_b[sl] = pk_v[j, sl] & 16383
            return carry
        lax.fori_loop(0, 8, fs, 0)

    def compute(j, rows_x, sc_x, di_x, sem_s, do_wait):
        @pl.when(do_wait)
        def _():
            pltpu.make_async_copy(sc_x, out_sh.at[di_x], sem_s).wait()
        for k in range(8):
            sl = pl.ds(k * 16, 16)
            v = pk_v[j, sl]
            di = lax.shift_right_logical(v, 14)
            di_x[sl] = di
            dg = plsc.load_gather(den_v, [di])
            al = w_v[j, sl] / dg
            ridx = lanes + (k * 16)

            def col_body(cb, carry):
                for cu in range(16):
                    cidx = cb * 16 + cu
                    cvec = jnp.full((16,), cidx, jnp.int32)
                    col = plsc.load_gather(rows_x, [ridx, cvec])
                    plsc.store_scatter(sc_x, [ridx, cvec], col * al)
                return carry
            lax.fori_loop(0, 4, col_body, 0)
        pltpu.async_copy(sc_x, out_sh.at[di_x], sem_s, add=True)

    fill_si(0, siA)
    pltpu.async_copy(xlh.at[siA], rowsA, semA)

    def body(t, carry):
        j0 = 2 * t
        fill_si(j0 + 1, siB)
        pltpu.async_copy(xlh.at[siB], rowsB, semB)
        pltpu.make_async_copy(xlh.at[siA], rowsA, semA).wait()
        compute(j0, rowsA, scA, diA, semSA, t > 0)

        @pl.when(t < (EROWS // 2 - 1))
        def _():
            fill_si(j0 + 2, siA)
            pltpu.async_copy(xlh.at[siA], rowsA, semA)
        pltpu.make_async_copy(xlh.at[siB], rowsB, semB).wait()
        compute(j0 + 1, rowsB, scB, diB, semSB, t > 0)
        return carry

    lax.fori_loop(0, EROWS // 2, body, 0)
    pltpu.make_async_copy(scA, out_sh.at[diA], semSA).wait()
    pltpu.make_async_copy(scB, out_sh.at[diB], semSB).wait()
    plsc.subcore_barrier()
    pltpu.sync_copy(out_sh.at[pl.ds(s * nslice, nslice)],
                    out2.at[pl.ds(s * nslice, nslice)])


@jax.jit
def _edge_call(edgeb, asrcp, adstp, xl):
    mesh1 = plsc.VectorSubcoreMesh(core_axis_name="c", subcore_axis_name="s",
                                   num_cores=1)
    f1 = pl.kernel(
        _edge1_body,
        out_type=[
            jax.ShapeDtypeStruct((SC_TILES, EROWS, 128), jnp.float32),
            jax.ShapeDtypeStruct((N_PAD,), jnp.float32),
        ],
        mesh=mesh1,
        compiler_params=pltpu.CompilerParams(needs_layout_passes=False,
                                             use_tc_tiling_on_sc=False),
        scratch_types=[
            pltpu.VMEM((EROWS, 128), jnp.int32),
            pltpu.VMEM((EROWS, 128), jnp.int32),
            pltpu.VMEM((EROWS, 128), jnp.float32),
            pltpu.VMEM((N_PAD,), jnp.float32),
            pltpu.VMEM((N_PAD,), jnp.float32),
            pltpu.VMEM((N_PAD,), jnp.float32),
            pltpu.VMEM_SHARED((N_PAD,), jnp.float32),
            pltpu.SemaphoreType.DMA,
        ],
    )
    w, den = f1(edgeb, asrcp, adstp)
    mesh2 = plsc.VectorSubcoreMesh(core_axis_name="c", subcore_axis_name="s",
                                   num_cores=1)
    f2 = pl.kernel(
        _edge2_body,
        out_type=jax.ShapeDtypeStruct((N_PAD, D_IN), jnp.float32),
        mesh=mesh2,
        compiler_params=pltpu.CompilerParams(needs_layout_passes=False,
                                             use_tc_tiling_on_sc=False),
        scratch_types=[
            pltpu.VMEM((EROWS, 128), jnp.int32),
            pltpu.VMEM((EROWS, 128), jnp.float32),
            pltpu.VMEM((N_PAD,), jnp.float32),
            pltpu.VMEM((128, D_IN), jnp.float32),
            pltpu.VMEM((128, D_IN), jnp.float32),
            pltpu.VMEM((128, D_IN), jnp.float32),
            pltpu.VMEM((128, D_IN), jnp.float32),
            pltpu.VMEM((128,), jnp.int32),
            pltpu.VMEM((128,), jnp.int32),
            pltpu.VMEM((128,), jnp.int32),
            pltpu.VMEM((128,), jnp.int32),
            pltpu.VMEM_SHARED((N_PAD, D_IN), jnp.float32),
            pltpu.SemaphoreType.DMA,
            pltpu.SemaphoreType.DMA,
            pltpu.SemaphoreType.DMA,
            pltpu.SemaphoreType.DMA,
        ],
    )
    return f2(edgeb, w, den, xl)


def _gat_edges(xl, aa, edge_index):
    loops = jnp.arange(N_OUT, dtype=jnp.int32)
    pad = E_PAD - E_RAW
    src = jnp.concatenate([edge_index[0].astype(jnp.int32), loops,
                           jnp.zeros((pad,), jnp.int32)])
    dst = jnp.concatenate([edge_index[1].astype(jnp.int32), loops,
                           jnp.full((pad,), PAD_DST, jnp.int32)])
    edgeb = (src | (dst << 14)).reshape(SC_TILES, EROWS, 128)
    asrcp = jnp.pad(aa[:, 0], (0, N_PAD - N_OUT))
    adstp = jnp.pad(aa[:, 1], (0, N_PAD - N_OUT))
    out2 = _edge_call(edgeb, asrcp, adstp, xl)
    return out2[:N_OUT]


def _final_body(o0_ref, gb_ref, ga_ref, bb_ref, out_ref):
    z = o0_ref[...] + gb_ref[...]
    z = _leaky(z)
    m = jnp.mean(z, axis=1, keepdims=True)
    c = z - m
    std = jnp.sqrt(jnp.sum(c * c, axis=1, keepdims=True) * (1.0 / (D_IN - 1)))
    out_ref[...] = ga_ref[...] * (c / (std + 1e-8)) + bb_ref[...]


def _final(o0, p, ga3, bb3):
    nblk = N_OUT // ROWS
    return pl.pallas_call(
        _final_body,
        grid=(nblk,),
        in_specs=[
            pl.BlockSpec((ROWS, D_IN), lambda i: (i, 0)),
            pl.BlockSpec((1, D_IN), lambda i: (0, 0)),
            pl.BlockSpec((1, D_IN), lambda i: (0, 0)),
            pl.BlockSpec((1, D_IN), lambda i: (0, 0)),
        ],
        out_specs=pl.BlockSpec((ROWS, D_IN), lambda i: (i, 0)),
        out_shape=jax.ShapeDtypeStruct((N_OUT, D_IN), jnp.float32),
    )(o0, p['gat_b'].reshape(1, -1), ga3, bb3)


def kernel(x, edge_index, style1, style2, params):
    p = params
    ga1, bb1, ga2, bb2, ga3, bb3 = _styles(style1, style2, p)
    u, sums = _stage1(x, p, ga1, bb1)
    xl, aa = _stage2(u, sums, p, ga2, bb2)
    o0 = _gat_edges(xl, aa, edge_index)
    return _final(o0, p, ga3, bb3)


# speedup vs baseline: 1.0080x; 1.0080x over previous
import functools

import jax
import jax.numpy as jnp
from jax import lax
from jax.experimental import pallas as pl
from jax.experimental.pallas import tpu as pltpu
from jax.experimental.pallas import tpu_sc as plsc

N_IN = 2048
D_IN = 64
D_HID = 256
N_OUT = 10000
ROWS = 1000

SC_CORES = 2
SC_TILES = 16
N_PAD = 10240
E_RAW = 320000 + N_OUT
EROWS = 162
E_PAD = SC_TILES * EROWS * 128
PAD_DST = N_OUT + 16


def _leaky(v):
    return jnp.where(v >= 0, v, 0.2 * v)


def _style_body(s1, s2, w11, w12, w13, w14, b11, b12, b13, b14,
                w21, w22, w23, w24, b21, b22, b23, b24,
                w31, w32, w33, w34, b31, b32, b33, b34,
                ga1, bb1, ga2, bb2, ga3, bb3):
    def mm(s, w):
        return lax.dot_general(s[...], w[...], (((1,), (1,)), ((), ())),
                               preferred_element_type=jnp.float32)
    ga1[...] = 0.5 * (mm(s1, w11) + b11[...]) + 0.5 * (mm(s2, w13) + b13[...])
    bb1[...] = 0.5 * (mm(s1, w12) + b12[...]) + 0.5 * (mm(s2, w14) + b14[...])
    ga2[...] = 0.5 * (mm(s1, w21) + b21[...]) + 0.5 * (mm(s2, w23) + b23[...])
    bb2[...] = 0.5 * (mm(s1, w22) + b22[...]) + 0.5 * (mm(s2, w24) + b24[...])
    ga3[...] = 0.5 * (mm(s1, w31) + b31[...]) + 0.5 * (mm(s2, w33) + b33[...])
    bb3[...] = 0.5 * (mm(s1, w32) + b32[...]) + 0.5 * (mm(s2, w34) + b34[...])


def _styles(s1, s2, p):
    args = [s1, s2]
    for tag in ('a1', 'a2', 'a3'):
        args += [p[tag + '_W1'], p[tag + '_W2'], p[tag + '_W3'], p[tag + '_W4'],
                 p[tag + '_b1'].reshape(1, -1), p[tag + '_b2'].reshape(1, -1),
                 p[tag + '_b3'].reshape(1, -1), p[tag + '_b4'].reshape(1, -1)]
    outs = [jax.ShapeDtypeStruct((1, 64), jnp.float32),
            jax.ShapeDtypeStruct((1, 64), jnp.float32),
            jax.ShapeDtypeStruct((1, 256), jnp.float32),
            jax.ShapeDtypeStruct((1, 256), jnp.float32),
            jax.ShapeDtypeStruct((1, 64), jnp.float32),
            jax.ShapeDtypeStruct((1, 64), jnp.float32)]
    return pl.pallas_call(_style_body, out_shape=outs)(*args)


def _stage1_body(w_ref, x_ref, trsb_ref, g_ref, b_ref, ga_ref, bb_ref,
                 fw_ref, fb_ref, u_ref, sums_ref):
    i = pl.program_id(0)
    t = lax.dot_general(w_ref[...], x_ref[...], (((1,), (0,)), ((), ())),
                        preferred_element_type=jnp.float32)
    t = t + trsb_ref[...]
    mu = jnp.mean(t, axis=1, keepdims=True)
    xc = t - mu
    var = jnp.mean(xc * xc, axis=1, keepdims=True)
    t = xc * lax.rsqrt(var + 1e-5) * g_ref[...] + b_ref[...]
    t = _leaky(t)
    m2 = jnp.mean(t, axis=1, keepdims=True)
    c2 = t - m2
    std = jnp.sqrt(jnp.sum(c2 * c2, axis=1, keepdims=True) * (1.0 / (D_IN - 1)))
    h = ga_ref[...] * (c2 / (std + 1e-8)) + bb_ref[...]
    u = lax.dot_general(h, fw_ref[...], (((1,), (1,)), ((), ())),
                        preferred_element_type=jnp.float32)
    u = u + fb_ref[...]
    u_ref[...] = u

    @pl.when(i == 0)
    def _():
        sums_ref[...] = jnp.zeros_like(sums_ref)
    sums_ref[0:1, :] += jnp.sum(u, axis=0, keepdims=True)
    sums_ref[1:2, :] += jnp.sum(u * u, axis=0, keepdims=True)


def _stage1(x, p, ga1, bb1):
    nblk = N_OUT // ROWS
    u, sums = pl.pallas_call(
        _stage1_body,
        grid=(nblk,),
        in_specs=[
            pl.BlockSpec((ROWS, N_IN), lambda i: (i, 0)),
            pl.BlockSpec((N_IN, D_IN), lambda i: (0, 0)),
            pl.BlockSpec((ROWS, 1), lambda i: (i, 0)),
            pl.BlockSpec((ROWS, 1), lambda i: (i, 0)),
            pl.BlockSpec((ROWS, 1), lambda i: (i, 0)),
            pl.BlockSpec((1, 64), lambda i: (0, 0)),
            pl.BlockSpec((1, 64), lambda i: (0, 0)),
            pl.BlockSpec((D_HID, D_IN), lambda i: (0, 0)),
            pl.BlockSpec((1, D_HID), lambda i: (0, 0)),
        ],
        out_specs=[
            pl.BlockSpec((ROWS, D_HID), lambda i: (i, 0)),
            pl.BlockSpec((8, D_HID), lambda i: (0, 0)),
        ],
        out_shape=[
            jax.ShapeDtypeStruct((N_OUT, D_HID), jnp.float32),
            jax.ShapeDtypeStruct((8, D_HID), jnp.float32),
        ],
    )(p['trs_W'], x, p['trs_b'].reshape(-1, 1), p['bn2_g'].reshape(-1, 1),
      p['bn2_b'].reshape(-1, 1), ga1, bb1, p['fc1_W'],
      p['fc1_b'].reshape(1, -1))
    return u, sums


def _stage2_body(u_ref, sums_ref, g_ref, b_ref, ga_ref, bb_ref, gw_ref,
                 av_ref, xl_ref, aa_ref):
    u = u_ref[...]
    s = sums_ref[...]
    mu = s[0:1, :] * (1.0 / N_OUT)
    ex2 = s[1:2, :] * (1.0 / N_OUT)
    var = ex2 - mu * mu
    h = (u - mu) * lax.rsqrt(var + 1e-5) * g_ref[...] + b_ref[...]
    h = _leaky(h)
    m = jnp.mean(h, axis=1, keepdims=True)
    c = h - m
    std = jnp.sqrt(jnp.sum(c * c, axis=1, keepdims=True) * (1.0 / (D_HID - 1)))
    h2 = ga_ref[...] * (c / (std + 1e-8)) + bb_ref[...]
    xl = lax.dot_general(h2, gw_ref[...], (((1,), (1,)), ((), ())),
                         preferred_element_type=jnp.float32)
    xl_ref[...] = xl
    aa_ref[...] = jnp.dot(xl, av_ref[...], preferred_element_type=jnp.float32)


def _stage2(u, sums, p, ga2, bb2):
    nblk = N_OUT // ROWS
    av = jnp.stack([p['gat_asrc'], p['gat_adst']], axis=1)
    xl, aa = pl.pallas_call(
        _stage2_body,
        grid=(nblk,),
        in_specs=[
            pl.BlockSpec((ROWS, D_HID), lambda i: (i, 0)),
            pl.BlockSpec((8, D_HID), lambda i: (0, 0)),
            pl.BlockSpec((1, D_HID), lambda i: (0, 0)),
            pl.BlockSpec((1, D_HID), lambda i: (0, 0)),
            pl.BlockSpec((1, D_HID), lambda i: (0, 0)),
            pl.BlockSpec((1, D_HID), lambda i: (0, 0)),
            pl.BlockSpec((D_IN, D_HID), lambda i: (0, 0)),
            pl.BlockSpec((D_IN, 2), lambda i: (0, 0)),
        ],
        out_specs=[
            pl.BlockSpec((ROWS, D_IN), lambda i: (i, 0)),
            pl.BlockSpec((ROWS, 2), lambda i: (i, 0)),
        ],
        out_shape=[
            jax.ShapeDtypeStruct((N_OUT, D_IN), jnp.float32),
            jax.ShapeDtypeStruct((N_OUT, 2), jnp.float32),
        ],
    )(u, sums, p['bn1_g'].reshape(1, -1), p['bn1_b'].reshape(1, -1),
      ga2, bb2, p['gat_W'], av)
    return xl, aa


def _edge1_body(edgeb, asrcp, adstp, wout, denout,
                pk_v, dst_v, w_v, asrc_v, adst_v, den_v, den_sh, sem):
    s = lax.axis_index("s")
    nslice = N_PAD // SC_TILES
    pltpu.sync_copy(edgeb.at[s], pk_v)
    pltpu.sync_copy(asrcp, asrc_v)
    pltpu.sync_copy(adstp, adst_v)

    zv = jnp.zeros((16,), jnp.float32)

    def zrow(j, carry):
        den_v[pl.ds(j * 16, 16)] = zv
        return carry
    lax.fori_loop(0, N_PAD // 16, zrow, 0)
    pltpu.sync_copy(den_v.at[pl.ds(s * nslice, nslice)],
                    den_sh.at[pl.ds(s * nslice, nslice)])

    def p1(j, carry):
        def p1c(k, carry2):
            sl = pl.ds(k * 16, 16)
            v = pk_v[j, sl]
            si = v & 16383
            di = lax.shift_right_logical(v, 14)
            dst_v[j, sl] = di
            e = plsc.load_gather(asrc_v, [si]) + plsc.load_gather(adst_v, [di])
            e = jnp.where(e >= 0, e, 0.2 * e)
            w_v[j, sl] = jnp.exp(e)
            return carry2
        return lax.fori_loop(0, 8, p1c, carry)
    lax.fori_loop(0, EROWS, p1, 0)
    plsc.subcore_barrier()

    def p2(j, carry):
        pltpu.sync_copy(w_v.at[j], den_sh.at[dst_v.at[j]], add=True)
        return carry
    lax.fori_loop(0, EROWS, p2, 0)
    plsc.subcore_barrier()
    pltpu.sync_copy(w_v, wout.at[s])
    pltpu.sync_copy(den_sh.at[pl.ds(s * nslice, nslice)],
                    denout.at[pl.ds(s * nslice, nslice)])


def _edge2_body(edgeb, wh, denh, xlh, out2,
                pk_v, w_v, den_v, rowsA, rowsB, scA, scB, siA, siB,
                diA, diB, out_sh, semA, semB, semSA, semSB):
    s = lax.axis_index("s")
    nslice = N_PAD // SC_TILES
    pltpu.sync_copy(edgeb.at[s], pk_v)
    pltpu.sync_copy(wh.at[s], w_v)
    pltpu.sync_copy(denh, den_v)

    zv = jnp.zeros((16,), jnp.float32)

    def zrow(j, carry):
        for c4 in range(4):
            scA[j, pl.ds(c4 * 16, 16)] = zv
        return carry
    lax.fori_loop(0, 128, zrow, 0)

    def zout(j, carry):
        pltpu.sync_copy(scA, out_sh.at[pl.ds(s * nslice + j * 128, 128)])
        return carry
    lax.fori_loop(0, nslice // 128, zout, 0)
    plsc.subcore_barrier()

    lanes = lax.iota(jnp.int32, 16)

    def fill_si(j, si_b):
        def fs(k, carry):
            sl = pl.ds(k * 16, 16)
            si_b[sl] = pk_v[j, sl] & 16383
            return carry
        lax.fori_loop(0, 8, fs, 0)

    def compute(j, rows_x, sc_x, di_x, sem_s, do_wait):
        @pl.when(do_wait)
        def _():
            pltpu.make_async_copy(sc_x, out_sh.at[di_x], sem_s).wait()
        for k in range(8):
            sl = pl.ds(k * 16, 16)
            v = pk_v[j, sl]
            di = lax.shift_right_logical(v, 14)
            di_x[sl] = di
            dg = plsc.load_gather(den_v, [di])
            al = w_v[j, sl] / dg
            ridx = lanes + (k * 16)

            def col_body(cc, carry):
                cidx = jnp.full((16,), cc, jnp.int32)
                col = plsc.load_gather(rows_x, [ridx, cidx])
                plsc.store_scatter(sc_x, [ridx, cidx], col * al)
                return carry
            lax.fori_loop(0, 64, col_body, 0)
        pltpu.async_copy(sc_x, out_sh.at[di_x], sem_s, add=True)

    fill_si(0, siA)
    pltpu.async_copy(xlh.at[siA], rowsA, semA)

    def body(t, carry):
        j0 = 2 * t
        fill_si(j0 + 1, siB)
        pltpu.async_copy(xlh.at[siB], rowsB, semB)
        pltpu.make_async_copy(xlh.at[siA], rowsA, semA).wait()
        compute(j0, rowsA, scA, diA, semSA, t > 0)

        @pl.when(t < (EROWS // 2 - 1))
        def _():
            fill_si(j0 + 2, siA)
            pltpu.async_copy(xlh.at[siA], rowsA, semA)
        pltpu.make_async_copy(xlh.at[siB], rowsB, semB).wait()
        compute(j0 + 1, rowsB, scB, diB, semSB, t > 0)
        return carry

    lax.fori_loop(0, EROWS // 2, body, 0)
    pltpu.make_async_copy(scA, out_sh.at[diA], semSA).wait()
    pltpu.make_async_copy(scB, out_sh.at[diB], semSB).wait()
    plsc.subcore_barrier()
    pltpu.sync_copy(out_sh.at[pl.ds(s * nslice, nslice)],
                    out2.at[pl.ds(s * nslice, nslice)])


@jax.jit
def _edge_call(edgeb, asrcp, adstp, xl):
    mesh1 = plsc.VectorSubcoreMesh(core_axis_name="c", subcore_axis_name="s",
                                   num_cores=1)
    f1 = pl.kernel(
        _edge1_body,
        out_type=[
            jax.ShapeDtypeStruct((SC_TILES, EROWS, 128), jnp.float32),
            jax.ShapeDtypeStruct((N_PAD,), jnp.float32),
        ],
        mesh=mesh1,
        compiler_params=pltpu.CompilerParams(needs_layout_passes=False,
                                             use_tc_tiling_on_sc=False),
        scratch_types=[
            pltpu.VMEM((EROWS, 128), jnp.int32),
            pltpu.VMEM((EROWS, 128), jnp.int32),
            pltpu.VMEM((EROWS, 128), jnp.float32),
            pltpu.VMEM((N_PAD,), jnp.float32),
            pltpu.VMEM((N_PAD,), jnp.float32),
            pltpu.VMEM((N_PAD,), jnp.float32),
            pltpu.VMEM_SHARED((N_PAD,), jnp.float32),
            pltpu.SemaphoreType.DMA,
        ],
    )
    w, den = f1(edgeb, asrcp, adstp)
    mesh2 = plsc.VectorSubcoreMesh(core_axis_name="c", subcore_axis_name="s",
                                   num_cores=1)
    f2 = pl.kernel(
        _edge2_body,
        out_type=jax.ShapeDtypeStruct((N_PAD, D_IN), jnp.float32),
        mesh=mesh2,
        compiler_params=pltpu.CompilerParams(needs_layout_passes=False,
                                             use_tc_tiling_on_sc=False),
        scratch_types=[
            pltpu.VMEM((EROWS, 128), jnp.int32),
            pltpu.VMEM((EROWS, 128), jnp.float32),
            pltpu.VMEM((N_PAD,), jnp.float32),
            pltpu.VMEM((128, D_IN), jnp.float32),
            pltpu.VMEM((128, D_IN), jnp.float32),
            pltpu.VMEM((128, D_IN), jnp.float32),
            pltpu.VMEM((128, D_IN), jnp.float32),
            pltpu.VMEM((128,), jnp.int32),
            pltpu.VMEM((128,), jnp.int32),
            pltpu.VMEM((128,), jnp.int32),
            pltpu.VMEM((128,), jnp.int32),
            pltpu.VMEM_SHARED((N_PAD, D_IN), jnp.float32),
            pltpu.SemaphoreType.DMA,
            pltpu.SemaphoreType.DMA,
            pltpu.SemaphoreType.DMA,
            pltpu.SemaphoreType.DMA,
        ],
    )
    return f2(edgeb, w, den, xl)


def _gat_edges(xl, aa, edge_index):
    loops = jnp.arange(N_OUT, dtype=jnp.int32)
    pad = E_PAD - E_RAW
    src = jnp.concatenate([edge_index[0].astype(jnp.int32), loops,
                           jnp.zeros((pad,), jnp.int32)])
    dst = jnp.concatenate([edge_index[1].astype(jnp.int32), loops,
                           jnp.full((pad,), PAD_DST, jnp.int32)])
    edgeb = (src | (dst << 14)).reshape(SC_TILES, EROWS, 128)
    asrcp = jnp.pad(aa[:, 0], (0, N_PAD - N_OUT))
    adstp = jnp.pad(aa[:, 1], (0, N_PAD - N_OUT))
    out2 = _edge_call(edgeb, asrcp, adstp, xl)
    return out2[:N_OUT]


def _final_body(o0_ref, gb_ref, ga_ref, bb_ref, out_ref):
    z = o0_ref[...] + gb_ref[...]
    z = _leaky(z)
    m = jnp.mean(z, axis=1, keepdims=True)
    c = z - m
    std = jnp.sqrt(jnp.sum(c * c, axis=1, keepdims=True) * (1.0 / (D_IN - 1)))
    out_ref[...] = ga_ref[...] * (c / (std + 1e-8)) + bb_ref[...]


def _final(o0, p, ga3, bb3):
    nblk = N_OUT // ROWS
    return pl.pallas_call(
        _final_body,
        grid=(nblk,),
        in_specs=[
            pl.BlockSpec((ROWS, D_IN), lambda i: (i, 0)),
            pl.BlockSpec((1, D_IN), lambda i: (0, 0)),
            pl.BlockSpec((1, D_IN), lambda i: (0, 0)),
            pl.BlockSpec((1, D_IN), lambda i: (0, 0)),
        ],
        out_specs=pl.BlockSpec((ROWS, D_IN), lambda i: (i, 0)),
        out_shape=jax.ShapeDtypeStruct((N_OUT, D_IN), jnp.float32),
    )(o0, p['gat_b'].reshape(1, -1), ga3, bb3)


def kernel(x, edge_index, style1, style2, params):
    p = params
    ga1, bb1, ga2, bb2, ga3, bb3 = _styles(style1, style2, p)
    u, sums = _stage1(x, p, ga1, bb1)
    xl, aa = _stage2(u, sums, p, ga2, bb2)
    o0 = _gat_edges(xl, aa, edge_index)
    return _final(o0, p, ga3, bb3)
